# Initial kernel scaffold; baseline (speedup 1.0000x reference)
import jax
import jax.numpy as jnp
from jax.experimental import pallas as pl


def kernel(features, index_map, entity_type, keys_weight, values_weight, size0, size1):
    raise NotImplementedError("write your pallas kernel here")


# speedup vs baseline: 3.3472x; 3.3472x over previous
import jax
import jax.numpy as jnp
from jax import lax
from jax.experimental import pallas as pl
from jax.experimental.pallas import tpu as pltpu
from jax.experimental.pallas import tpu_sc as plsc

_D = 64
_POS = 441
_EXT = 10.0
_STR0, _STR1 = 1.0, 21.0
_NC, _NS = 2, 16
_NW = _NC * _NS
_N = 1024
_SEQ = 256
_NB = _N // _SEQ
_RPW = _N // _NW
_WPB = _SEQ // _RPW
_CH = _SEQ // 16


def _body(feat_h, imap_h, et_h, zoff_h, kw_h, vw_h, outk_h, outv_h,
          feat_v, imap_v, x_v, y_v, eto_v, zoff_v,
          kidx0_v, kidx1_v, vidx0_v, vidx1_v, krows_v, vrows_v, sem):
    wid = lax.axis_index("s") * _NC + lax.axis_index("c")
    b = wid // _WPB
    iloc0 = (wid % _WPB) * _RPW
    base = b * _SEQ

    pltpu.sync_copy(feat_h, feat_v)
    pltpu.sync_copy(imap_h.at[pl.ds(base, _SEQ)], imap_v)
    pltpu.sync_copy(et_h.at[pl.ds(base, _SEQ)], eto_v)
    pltpu.sync_copy(zoff_h, zoff_v)

    for c in range(_CH):
        sl = pl.ds(c * 16, 16)
        jv = imap_v[sl] * 8
        x_v[sl] = plsc.load_gather(feat_v, [jv])
        y_v[sl] = plsc.load_gather(feat_v, [jv + 1])
        eto_v[sl] = eto_v[sl] * _POS

    zoff = zoff_v[...]

    def row(t, ii):
        isp = lax.broadcast_in_dim(ii, (16,), ())
        xi = plsc.load_gather(x_v, [isp])
        yi = plsc.load_gather(y_v, [isp])
        for c in range(_CH):
            sl = pl.ds(c * 16, 16)
            dx = x_v[sl] - xi
            dy = y_v[sl] - yi
            dxc = jnp.maximum(jnp.minimum(dx, _EXT), -_EXT)
            dyc = jnp.maximum(jnp.minimum(dy, _EXT), -_EXT)
            fidx = (dxc + _EXT) * _STR0 + (dyc + _EXT) * _STR1
            iidx = fidx.astype(jnp.int32) + zoff
            csl = pl.ds((c % 8) * 16, 16)
            if c < 8:
                kidx0_v[csl] = iidx
                vidx0_v[csl] = iidx + eto_v[sl]
            else:
                kidx1_v[csl] = iidx
                vidx1_v[csl] = iidx + eto_v[sl]
        cps = [
            pltpu.async_copy(kw_h.at[kidx0_v], krows_v.at[pl.ds(0, 128)], sem),
            pltpu.async_copy(kw_h.at[kidx1_v], krows_v.at[pl.ds(128, 128)], sem),
            pltpu.async_copy(vw_h.at[vidx0_v], vrows_v.at[pl.ds(0, 128)], sem),
            pltpu.async_copy(vw_h.at[vidx1_v], vrows_v.at[pl.ds(128, 128)], sem),
        ]
        for cp in cps:
            cp.wait()
        r = b * _SEQ + ii
        pltpu.sync_copy(krows_v, outk_h.at[r])
        pltpu.sync_copy(vrows_v, outv_h.at[r])
        return ii + 1

    lax.fori_loop(0, _RPW, row, iloc0)


def kernel(features, index_map, entity_type, keys_weight, values_weight,
           size0, size1):
    n = features.shape[0]
    imap32 = index_map.astype(jnp.int32)
    et32 = entity_type.astype(jnp.int32)
    zoff = (jnp.asarray(size0, jnp.int32) * jnp.asarray(size1, jnp.int32)
            - jnp.int32(n))
    zoff16 = jnp.zeros((16,), jnp.int32) + zoff

    mesh = plsc.VectorSubcoreMesh(core_axis_name="c", subcore_axis_name="s",
                                  num_cores=_NC, num_subcores=_NS)
    run = pl.kernel(
        _body,
        out_type=(jax.ShapeDtypeStruct((n, _SEQ, _D), jnp.float32),
                  jax.ShapeDtypeStruct((n, _SEQ, _D), jnp.float32)),
        mesh=mesh,
        compiler_params=pltpu.CompilerParams(needs_layout_passes=False,
                                             use_tc_tiling_on_sc=False),
        scratch_types=[
            pltpu.VMEM((n * 8,), jnp.float32),
            pltpu.VMEM((_SEQ,), jnp.int32),
            pltpu.VMEM((_SEQ,), jnp.float32),
            pltpu.VMEM((_SEQ,), jnp.float32),
            pltpu.VMEM((_SEQ,), jnp.int32),
            pltpu.VMEM((16,), jnp.int32),
            pltpu.VMEM((128,), jnp.int32),
            pltpu.VMEM((128,), jnp.int32),
            pltpu.VMEM((128,), jnp.int32),
            pltpu.VMEM((128,), jnp.int32),
            pltpu.VMEM((_SEQ, _D), jnp.float32),
            pltpu.VMEM((_SEQ, _D), jnp.float32),
            pltpu.SemaphoreType.DMA,
        ],
    )
    ok, ov = run(features.reshape(-1), imap32, et32, zoff16,
                 keys_weight, values_weight)
    s1 = n // _NB
    return (ok.reshape(_NB, s1, _SEQ, _D), ov.reshape(_NB, s1, _SEQ, _D))


# speedup vs baseline: 3.3780x; 1.0092x over previous
import jax
import jax.numpy as jnp
from jax import lax
from jax.experimental import pallas as pl
from jax.experimental.pallas import tpu as pltpu
from jax.experimental.pallas import tpu_sc as plsc

_D = 64
_POS = 441
_EXT = 10.0
_STR0, _STR1 = 1.0, 21.0
_NC, _NS = 2, 16
_NW = _NC * _NS
_N = 1024
_SEQ = 256
_NB = _N // _SEQ
_RPW = _N // _NW
_WPB = _SEQ // _RPW
_CH = _SEQ // 16


def _body(feat_h, imap_h, et_h, zoff_h, kw_h, vw_h, outk_h, outv_h,
          feat_v, imap_v, x_v, y_v, eto_v, zoff_v,
          kidx_a, vidx_a, kidx_b, vidx_b,
          krows_a, vrows_a, krows_b, vrows_b,
          gsem_a, gsem_b, osem_a, osem_b):
    wid = lax.axis_index("s") * _NC + lax.axis_index("c")
    b = wid // _WPB
    iloc0 = (wid % _WPB) * _RPW
    base = b * _SEQ

    pltpu.sync_copy(feat_h, feat_v)
    pltpu.sync_copy(imap_h.at[pl.ds(base, _SEQ)], imap_v)
    pltpu.sync_copy(et_h.at[pl.ds(base, _SEQ)], eto_v)
    pltpu.sync_copy(zoff_h, zoff_v)

    for c in range(_CH):
        sl = pl.ds(c * 16, 16)
        jv = imap_v[sl] * 8
        x_v[sl] = plsc.load_gather(feat_v, [jv])
        y_v[sl] = plsc.load_gather(feat_v, [jv + 1])
        eto_v[sl] = eto_v[sl] * _POS

    zoff = zoff_v[...]

    def compute_idx(ii, kidx, vidx):
        isp = lax.broadcast_in_dim(jnp.minimum(ii, _SEQ - 1), (16,), ())
        xi = plsc.load_gather(x_v, [isp])
        yi = plsc.load_gather(y_v, [isp])
        for c in range(_CH):
            sl = pl.ds(c * 16, 16)
            dx = x_v[sl] - xi
            dy = y_v[sl] - yi
            dxc = jnp.maximum(jnp.minimum(dx, _EXT), -_EXT)
            dyc = jnp.maximum(jnp.minimum(dy, _EXT), -_EXT)
            fidx = (dxc + _EXT) * _STR0 + (dyc + _EXT) * _STR1
            iidx = fidx.astype(jnp.int32) + zoff
            kidx[c // 8, pl.ds((c % 8) * 16, 16)] = iidx
            vidx[c // 8, pl.ds((c % 8) * 16, 16)] = iidx + eto_v[sl]

    def fire_gathers(kidx, vidx, krows, vrows, sem):
        h0, h1 = jnp.int32(0), jnp.int32(1)
        return [
            pltpu.async_copy(kw_h.at[kidx.at[h0]], krows.at[pl.ds(0, 128)], sem),
            pltpu.async_copy(kw_h.at[kidx.at[h1]], krows.at[pl.ds(128, 128)], sem),
            pltpu.async_copy(vw_h.at[vidx.at[h0]], vrows.at[pl.ds(0, 128)], sem),
            pltpu.async_copy(vw_h.at[vidx.at[h1]], vrows.at[pl.ds(128, 128)], sem),
        ]

    def drain_gathers(krows, vrows, sem):
        for half in range(2):
            hs = pl.ds(half * 128, 128)
            pltpu.make_async_copy(kw_h.at[pl.ds(0, 128)], krows.at[hs], sem).wait()
            pltpu.make_async_copy(vw_h.at[pl.ds(0, 128)], vrows.at[hs], sem).wait()

    def fire_outputs(r, krows, vrows, sem):
        return [pltpu.async_copy(krows, outk_h.at[r], sem),
                pltpu.async_copy(vrows, outv_h.at[r], sem)]

    def drain_outputs(krows, vrows, sem):
        r0 = jnp.int32(0)
        pltpu.make_async_copy(krows, outk_h.at[r0], sem).wait()
        pltpu.make_async_copy(vrows, outv_h.at[r0], sem).wait()

    compute_idx(iloc0, kidx_a, vidx_a)
    fire_gathers(kidx_a, vidx_a, krows_a, vrows_a, gsem_a)

    def step(u, ii):
        compute_idx(ii + 1, kidx_b, vidx_b)
        drain_gathers(krows_a, vrows_a, gsem_a)
        pl.when(ii > iloc0)(
            lambda: drain_outputs(krows_b, vrows_b, osem_b))
        out_a = fire_outputs(b * _SEQ + ii, krows_a, vrows_a, osem_a)
        g_b = fire_gathers(kidx_b, vidx_b, krows_b, vrows_b, gsem_b)
        compute_idx(ii + 2, kidx_a, vidx_a)
        for d in g_b:
            d.wait()
        fire_outputs(b * _SEQ + ii + 1, krows_b, vrows_b, osem_b)
        for d in out_a:
            d.wait()
        @pl.when(ii < iloc0 + (_RPW - 2))
        def _():
            fire_gathers(kidx_a, vidx_a, krows_a, vrows_a, gsem_a)
        return ii + 2

    lax.fori_loop(0, _RPW // 2, step, iloc0)
    drain_outputs(krows_b, vrows_b, osem_b)


def kernel(features, index_map, entity_type, keys_weight, values_weight,
           size0, size1):
    n = features.shape[0]
    imap32 = index_map.astype(jnp.int32)
    et32 = entity_type.astype(jnp.int32)
    zoff = (jnp.asarray(size0, jnp.int32) * jnp.asarray(size1, jnp.int32)
            - jnp.int32(n))
    zoff16 = jnp.zeros((16,), jnp.int32) + zoff

    mesh = plsc.VectorSubcoreMesh(core_axis_name="c", subcore_axis_name="s",
                                  num_cores=_NC, num_subcores=_NS)
    run = pl.kernel(
        _body,
        out_type=(jax.ShapeDtypeStruct((n, _SEQ, _D), jnp.float32),
                  jax.ShapeDtypeStruct((n, _SEQ, _D), jnp.float32)),
        mesh=mesh,
        compiler_params=pltpu.CompilerParams(needs_layout_passes=False,
                                             use_tc_tiling_on_sc=False),
        scratch_types=[
            pltpu.VMEM((n * 8,), jnp.float32),
            pltpu.VMEM((_SEQ,), jnp.int32),
            pltpu.VMEM((_SEQ,), jnp.float32),
            pltpu.VMEM((_SEQ,), jnp.float32),
            pltpu.VMEM((_SEQ,), jnp.int32),
            pltpu.VMEM((16,), jnp.int32),
            pltpu.VMEM((2, 128), jnp.int32),
            pltpu.VMEM((2, 128), jnp.int32),
            pltpu.VMEM((2, 128), jnp.int32),
            pltpu.VMEM((2, 128), jnp.int32),
            pltpu.VMEM((_SEQ, _D), jnp.float32),
            pltpu.VMEM((_SEQ, _D), jnp.float32),
            pltpu.VMEM((_SEQ, _D), jnp.float32),
            pltpu.VMEM((_SEQ, _D), jnp.float32),
            pltpu.SemaphoreType.DMA,
            pltpu.SemaphoreType.DMA,
            pltpu.SemaphoreType.DMA,
            pltpu.SemaphoreType.DMA,
        ],
    )
    ok, ov = run(features.reshape(-1), imap32, et32, zoff16,
                 keys_weight, values_weight)
    s1 = n // _NB
    return (ok.reshape(_NB, s1, _SEQ, _D), ov.reshape(_NB, s1, _SEQ, _D))
